# baseline (speedup 1.0000x reference)
import functools

import jax
import jax.numpy as jnp
from jax import lax
from jax.experimental import pallas as pl
from jax.experimental.pallas import tpu as pltpu
from jax.experimental.pallas import tpu_sc as plsc

NUM_STATE = 1000
NUM_ACTION = 64
BATCH = 16384

_info = plsc.get_sparse_core_info()
_NC = _info.num_cores
_NS = _info.num_subcores
_NW = _NC * _NS
_BPW = BATCH // _NW
_CHUNK = 128
_NCHUNK = _BPW // _CHUNK


@functools.partial(
    pl.pallas_call,
    out_shape=jax.ShapeDtypeStruct((NUM_STATE, NUM_ACTION), jnp.float32),
)
def _fuse_bias(wt_ref, b_ref, o_ref):
    o_ref[...] = wt_ref[...] + b_ref[...]


@functools.partial(
    pl.kernel,
    out_type=jax.ShapeDtypeStruct((BATCH, NUM_ACTION), jnp.float32),
    mesh=plsc.VectorSubcoreMesh(core_axis_name="c", subcore_axis_name="s"),
    scratch_types=[
        pltpu.VMEM((_BPW,), jnp.int32),
        pltpu.VMEM((_BPW, NUM_ACTION), jnp.float32),
        pltpu.SemaphoreType.DMA,
        pltpu.SemaphoreType.DMA,
    ],
    compiler_params=pltpu.CompilerParams(use_tc_tiling_on_sc=False),
)
def _qnet_gather(x_hbm, wtb_hbm, out_hbm, idx_v, rows_v, gsem, ssem):
    wid = lax.axis_index("s") * _NC + lax.axis_index("c")
    base = wid * _BPW

    pltpu.sync_copy(x_hbm.at[pl.ds(base, _BPW)], idx_v)

    copies = [
        pltpu.async_copy(
            wtb_hbm.at[idx_v.at[pl.ds(j * _CHUNK, _CHUNK)]],
            rows_v.at[pl.ds(j * _CHUNK, _CHUNK)],
            gsem,
        )
        for j in range(_NCHUNK)
    ]

    stores = []
    for j in range(_NCHUNK):
        copies[j].wait()
        lo = j * _CHUNK
        stores.append(
            pltpu.async_copy(
                rows_v.at[pl.ds(lo, _CHUNK)],
                out_hbm.at[pl.ds(base + lo, _CHUNK)],
                ssem,
            )
        )
    for s in stores:
        s.wait()


def kernel(x, W, b):
    wtb = _fuse_bias(jnp.transpose(W), b.reshape(1, NUM_ACTION))
    return _qnet_gather(x.astype(jnp.int32), wtb)
